# baseline (device time: 11181 ns/iter reference)
import jax
import jax.numpy as jnp
from jax import lax
from jax.experimental import pallas as pl
from jax.experimental.pallas import tpu as pltpu

N_DEV = 8
N_HALF = 1
N_RCH = 8


def kernel(x, w_mat):
    m_per, k = x.shape
    n = w_mat.shape[1]
    n_per = n // N_DEV
    n_half = n // N_HALF
    k_per = k // N_RCH

    def body(x_hbm, w_hbm, out_hbm, xv_ref, wr_ref, acc_ref, send_ref,
             recv_ref, stage_ref, xcopy_sem, wcopy_sems, store_sems,
             send_sems, recv_sems):
        my = lax.axis_index("i")

        barrier_sem = pltpu.get_barrier_semaphore()
        for h in range(1, N_DEV):
            nbr = lax.rem(my + h, N_DEV)
            pl.semaphore_signal(
                barrier_sem, inc=1,
                device_id=(nbr,), device_id_type=pl.DeviceIdType.MESH,
            )

        xcopy = pltpu.make_async_copy(x_hbm, xv_ref, xcopy_sem)
        xcopy.start()
        wcopies = []
        for half in range(N_HALF):
            for c in range(N_RCH):
                i = half * N_RCH + c
                cp = pltpu.make_async_copy(
                    w_hbm.at[pl.ds(c * k_per, k_per),
                             pl.ds(half * n_half, n_half)],
                    wr_ref.at[i], wcopy_sems.at[i],
                )
                cp.start()
                wcopies.append(cp)

        xcopy.wait()
        xv = xv_ref[...].astype(jnp.bfloat16)

        rdmas = []
        stores = []

        def send_block(t):
            blk = acc_ref[:, t * n_per:(t + 1) * n_per]
            act = blk * jax.nn.sigmoid(blk)

            @pl.when(t == my)
            def _():
                stage_ref[N_DEV - 1] = act
            own_store = pltpu.make_async_copy(
                stage_ref.at[N_DEV - 1],
                out_hbm.at[pl.ds(my * m_per, m_per), :],
                store_sems.at[N_DEV - 1],
            )

            @pl.when(t == my)
            def _():
                own_store.start()
            stores.append((t == my, own_store))

            @pl.when(t != my)
            def _():
                send_ref[t] = act.astype(jnp.bfloat16)
            rdma = pltpu.make_async_remote_copy(
                src_ref=send_ref.at[t],
                dst_ref=recv_ref.at[my],
                send_sem=send_sems.at[t],
                recv_sem=recv_sems.at[my],
                device_id=(t,),
                device_id_type=pl.DeviceIdType.MESH,
            )

            @pl.when(t != my)
            def _():
                rdma.start()
            rdmas.append((t != my, rdma))

        first = True
        for half in range(N_HALF):
            for c in range(N_RCH):
                i = half * N_RCH + c
                wcopies[i].wait()
                wb = wr_ref[i].astype(jnp.bfloat16)
                part = jnp.dot(
                    xv[:, c * k_per:(c + 1) * k_per], wb,
                    preferred_element_type=jnp.float32,
                )
                sl = pl.ds(half * n_half, n_half)
                if c == 0:
                    acc_ref[:, sl] = part
                else:
                    acc_ref[:, sl] = acc_ref[:, sl] + part

            if first:
                pl.semaphore_wait(barrier_sem, N_DEV - 1)
                first = False
            for t in range(half * N_DEV // N_HALF,
                           (half + 1) * N_DEV // N_HALF):
                send_block(t)

        for h in range(1, N_DEV):
            src = lax.rem(my + N_DEV - h, N_DEV)
            recv = pltpu.make_async_remote_copy(
                src_ref=send_ref.at[0],
                dst_ref=recv_ref.at[src],
                send_sem=send_sems.at[0],
                recv_sem=recv_sems.at[src],
                device_id=(0,),
                device_id_type=pl.DeviceIdType.MESH,
            )
            recv.wait_recv()
            stage_ref[h - 1] = recv_ref[src].astype(jnp.float32)
            store = pltpu.make_async_copy(
                stage_ref.at[h - 1],
                out_hbm.at[pl.ds(src * m_per, m_per), :],
                store_sems.at[h - 1],
            )
            store.start()
            stores.append((None, store))

        for pred, store in stores:
            if pred is None:
                store.wait()
            else:
                @pl.when(pred)
                def _():
                    store.wait()
        for pred, rdma in rdmas:
            @pl.when(pred)
            def _():
                rdma.wait_send()

    out_shape = jax.ShapeDtypeStruct((N_DEV * m_per, n_per), jnp.float32)
    return pl.pallas_call(
        body,
        out_shape=out_shape,
        in_specs=[
            pl.BlockSpec(memory_space=pl.ANY),
            pl.BlockSpec(memory_space=pl.ANY),
        ],
        out_specs=pl.BlockSpec(memory_space=pltpu.MemorySpace.HBM),
        scratch_shapes=[
            pltpu.VMEM((m_per, k), jnp.float32),
            pltpu.VMEM((N_HALF * N_RCH, k_per, n_half), jnp.float32),
            pltpu.VMEM((m_per, n), jnp.float32),
            pltpu.VMEM((N_DEV, m_per, n_per), jnp.bfloat16),
            pltpu.VMEM((N_DEV, m_per, n_per), jnp.bfloat16),
            pltpu.VMEM((N_DEV, m_per, n_per), jnp.float32),
            pltpu.SemaphoreType.DMA,
            pltpu.SemaphoreType.DMA((N_HALF * N_RCH,)),
            pltpu.SemaphoreType.DMA((N_DEV,)),
            pltpu.SemaphoreType.DMA((N_DEV,)),
            pltpu.SemaphoreType.DMA((N_DEV,)),
        ],
        compiler_params=pltpu.CompilerParams(collective_id=0),
    )(
        pltpu.with_memory_space_constraint(x, pltpu.MemorySpace.HBM),
        pltpu.with_memory_space_constraint(w_mat, pltpu.MemorySpace.HBM),
    )


# device time: 10513 ns/iter; 1.0635x vs baseline; 1.0635x over previous
import jax
import jax.numpy as jnp
from jax import lax
from jax.experimental import pallas as pl
from jax.experimental.pallas import tpu as pltpu

N_DEV = 8
N_HALF = 4
N_RCH = 2


def kernel(x, w_mat):
    m_per, k = x.shape
    n = w_mat.shape[1]
    n_per = n // N_DEV
    n_half = n // N_HALF
    k_per = k // N_RCH

    def body(x_hbm, w_hbm, out_hbm, xv_ref, wr_ref, acc_ref, send_ref,
             recv_ref, stage_ref, xcopy_sem, wcopy_sems, store_sems,
             send_sems, recv_sems):
        my = lax.axis_index("i")

        barrier_sem = pltpu.get_barrier_semaphore()
        for h in range(1, N_DEV):
            nbr = lax.rem(my + h, N_DEV)
            pl.semaphore_signal(
                barrier_sem, inc=1,
                device_id=(nbr,), device_id_type=pl.DeviceIdType.MESH,
            )

        xcopy = pltpu.make_async_copy(x_hbm, xv_ref, xcopy_sem)
        xcopy.start()
        wcopies = []
        for half in range(N_HALF):
            for c in range(N_RCH):
                i = half * N_RCH + c
                cp = pltpu.make_async_copy(
                    w_hbm.at[pl.ds(c * k_per, k_per),
                             pl.ds(half * n_half, n_half)],
                    wr_ref.at[i], wcopy_sems.at[i],
                )
                cp.start()
                wcopies.append(cp)

        xcopy.wait()
        xv = xv_ref[...].astype(jnp.bfloat16)

        rdmas = []
        stores = []

        def send_block(t):
            blk = acc_ref[:, t * n_per:(t + 1) * n_per]
            act = blk * jax.nn.sigmoid(blk)

            @pl.when(t == my)
            def _():
                stage_ref[N_DEV - 1] = act
            own_store = pltpu.make_async_copy(
                stage_ref.at[N_DEV - 1],
                out_hbm.at[pl.ds(my * m_per, m_per), :],
                store_sems.at[N_DEV - 1],
            )

            @pl.when(t == my)
            def _():
                own_store.start()
            stores.append((t == my, own_store))

            @pl.when(t != my)
            def _():
                send_ref[t] = act.astype(jnp.bfloat16)
            rdma = pltpu.make_async_remote_copy(
                src_ref=send_ref.at[t],
                dst_ref=recv_ref.at[my],
                send_sem=send_sems.at[t],
                recv_sem=recv_sems.at[my],
                device_id=(t,),
                device_id_type=pl.DeviceIdType.MESH,
            )

            @pl.when(t != my)
            def _():
                rdma.start()
            rdmas.append((t != my, rdma))

        first = True
        for half in range(N_HALF):
            for c in range(N_RCH):
                i = half * N_RCH + c
                wcopies[i].wait()
                wb = wr_ref[i].astype(jnp.bfloat16)
                part = jnp.dot(
                    xv[:, c * k_per:(c + 1) * k_per], wb,
                    preferred_element_type=jnp.float32,
                )
                sl = pl.ds(half * n_half, n_half)
                if c == 0:
                    acc_ref[:, sl] = part
                else:
                    acc_ref[:, sl] = acc_ref[:, sl] + part

            if first:
                pl.semaphore_wait(barrier_sem, N_DEV - 1)
                first = False
            for t in range(half * N_DEV // N_HALF,
                           (half + 1) * N_DEV // N_HALF):
                send_block(t)

        for h in range(1, N_DEV):
            src = lax.rem(my + N_DEV - h, N_DEV)
            recv = pltpu.make_async_remote_copy(
                src_ref=send_ref.at[0],
                dst_ref=recv_ref.at[src],
                send_sem=send_sems.at[0],
                recv_sem=recv_sems.at[src],
                device_id=(0,),
                device_id_type=pl.DeviceIdType.MESH,
            )
            recv.wait_recv()
            stage_ref[h - 1] = recv_ref[src].astype(jnp.float32)
            store = pltpu.make_async_copy(
                stage_ref.at[h - 1],
                out_hbm.at[pl.ds(src * m_per, m_per), :],
                store_sems.at[h - 1],
            )
            store.start()
            stores.append((None, store))

        for pred, store in stores:
            if pred is None:
                store.wait()
            else:
                @pl.when(pred)
                def _():
                    store.wait()
        for pred, rdma in rdmas:
            @pl.when(pred)
            def _():
                rdma.wait_send()

    out_shape = jax.ShapeDtypeStruct((N_DEV * m_per, n_per), jnp.float32)
    return pl.pallas_call(
        body,
        out_shape=out_shape,
        in_specs=[
            pl.BlockSpec(memory_space=pl.ANY),
            pl.BlockSpec(memory_space=pl.ANY),
        ],
        out_specs=pl.BlockSpec(memory_space=pltpu.MemorySpace.HBM),
        scratch_shapes=[
            pltpu.VMEM((m_per, k), jnp.float32),
            pltpu.VMEM((N_HALF * N_RCH, k_per, n_half), jnp.float32),
            pltpu.VMEM((m_per, n), jnp.float32),
            pltpu.VMEM((N_DEV, m_per, n_per), jnp.bfloat16),
            pltpu.VMEM((N_DEV, m_per, n_per), jnp.bfloat16),
            pltpu.VMEM((N_DEV, m_per, n_per), jnp.float32),
            pltpu.SemaphoreType.DMA,
            pltpu.SemaphoreType.DMA((N_HALF * N_RCH,)),
            pltpu.SemaphoreType.DMA((N_DEV,)),
            pltpu.SemaphoreType.DMA((N_DEV,)),
            pltpu.SemaphoreType.DMA((N_DEV,)),
        ],
        compiler_params=pltpu.CompilerParams(collective_id=0),
    )(
        pltpu.with_memory_space_constraint(x, pltpu.MemorySpace.HBM),
        pltpu.with_memory_space_constraint(w_mat, pltpu.MemorySpace.HBM),
    )


# device time: 10397 ns/iter; 1.0754x vs baseline; 1.0112x over previous
import jax
import jax.numpy as jnp
from jax import lax
from jax.experimental import pallas as pl
from jax.experimental.pallas import tpu as pltpu

N_DEV = 8
N_HALF = 8
N_RCH = 1


def kernel(x, w_mat):
    m_per, k = x.shape
    n = w_mat.shape[1]
    n_per = n // N_DEV
    n_half = n // N_HALF
    k_per = k // N_RCH

    def body(x_hbm, w_hbm, out_hbm, xv_ref, wr_ref, acc_ref, send_ref,
             recv_ref, stage_ref, xcopy_sem, wcopy_sems, store_sems,
             send_sems, recv_sems):
        my = lax.axis_index("i")

        barrier_sem = pltpu.get_barrier_semaphore()
        for h in range(1, N_DEV):
            nbr = lax.rem(my + h, N_DEV)
            pl.semaphore_signal(
                barrier_sem, inc=1,
                device_id=(nbr,), device_id_type=pl.DeviceIdType.MESH,
            )

        xcopy = pltpu.make_async_copy(x_hbm, xv_ref, xcopy_sem)
        xcopy.start()
        wcopies = []
        for half in range(N_HALF):
            for c in range(N_RCH):
                i = half * N_RCH + c
                cp = pltpu.make_async_copy(
                    w_hbm.at[pl.ds(c * k_per, k_per),
                             pl.ds(half * n_half, n_half)],
                    wr_ref.at[i], wcopy_sems.at[i],
                )
                cp.start()
                wcopies.append(cp)

        xcopy.wait()
        xv = xv_ref[...].astype(jnp.bfloat16)

        rdmas = []
        stores = []

        def send_block(t):
            blk = acc_ref[:, t * n_per:(t + 1) * n_per]
            act = blk * jax.nn.sigmoid(blk)

            @pl.when(t == my)
            def _():
                stage_ref[N_DEV - 1] = act
            own_store = pltpu.make_async_copy(
                stage_ref.at[N_DEV - 1],
                out_hbm.at[pl.ds(my * m_per, m_per), :],
                store_sems.at[N_DEV - 1],
            )

            @pl.when(t == my)
            def _():
                own_store.start()
            stores.append((t == my, own_store))

            @pl.when(t != my)
            def _():
                send_ref[t] = act.astype(jnp.bfloat16)
            rdma = pltpu.make_async_remote_copy(
                src_ref=send_ref.at[t],
                dst_ref=recv_ref.at[my],
                send_sem=send_sems.at[t],
                recv_sem=recv_sems.at[my],
                device_id=(t,),
                device_id_type=pl.DeviceIdType.MESH,
            )

            @pl.when(t != my)
            def _():
                rdma.start()
            rdmas.append((t != my, rdma))

        first = True
        for half in range(N_HALF):
            for c in range(N_RCH):
                i = half * N_RCH + c
                wcopies[i].wait()
                wb = wr_ref[i].astype(jnp.bfloat16)
                part = jnp.dot(
                    xv[:, c * k_per:(c + 1) * k_per], wb,
                    preferred_element_type=jnp.float32,
                )
                sl = pl.ds(half * n_half, n_half)
                if c == 0:
                    acc_ref[:, sl] = part
                else:
                    acc_ref[:, sl] = acc_ref[:, sl] + part

            if first:
                pl.semaphore_wait(barrier_sem, N_DEV - 1)
                first = False
            for t in range(half * N_DEV // N_HALF,
                           (half + 1) * N_DEV // N_HALF):
                send_block(t)

        for h in range(1, N_DEV):
            src = lax.rem(my + N_DEV - h, N_DEV)
            recv = pltpu.make_async_remote_copy(
                src_ref=send_ref.at[0],
                dst_ref=recv_ref.at[src],
                send_sem=send_sems.at[0],
                recv_sem=recv_sems.at[src],
                device_id=(0,),
                device_id_type=pl.DeviceIdType.MESH,
            )
            recv.wait_recv()
            stage_ref[h - 1] = recv_ref[src].astype(jnp.float32)
            store = pltpu.make_async_copy(
                stage_ref.at[h - 1],
                out_hbm.at[pl.ds(src * m_per, m_per), :],
                store_sems.at[h - 1],
            )
            store.start()
            stores.append((None, store))

        for pred, store in stores:
            if pred is None:
                store.wait()
            else:
                @pl.when(pred)
                def _():
                    store.wait()
        for pred, rdma in rdmas:
            @pl.when(pred)
            def _():
                rdma.wait_send()

    out_shape = jax.ShapeDtypeStruct((N_DEV * m_per, n_per), jnp.float32)
    return pl.pallas_call(
        body,
        out_shape=out_shape,
        in_specs=[
            pl.BlockSpec(memory_space=pl.ANY),
            pl.BlockSpec(memory_space=pl.ANY),
        ],
        out_specs=pl.BlockSpec(memory_space=pltpu.MemorySpace.HBM),
        scratch_shapes=[
            pltpu.VMEM((m_per, k), jnp.float32),
            pltpu.VMEM((N_HALF * N_RCH, k_per, n_half), jnp.float32),
            pltpu.VMEM((m_per, n), jnp.float32),
            pltpu.VMEM((N_DEV, m_per, n_per), jnp.bfloat16),
            pltpu.VMEM((N_DEV, m_per, n_per), jnp.bfloat16),
            pltpu.VMEM((N_DEV, m_per, n_per), jnp.float32),
            pltpu.SemaphoreType.DMA,
            pltpu.SemaphoreType.DMA((N_HALF * N_RCH,)),
            pltpu.SemaphoreType.DMA((N_DEV,)),
            pltpu.SemaphoreType.DMA((N_DEV,)),
            pltpu.SemaphoreType.DMA((N_DEV,)),
        ],
        compiler_params=pltpu.CompilerParams(collective_id=0),
    )(
        pltpu.with_memory_space_constraint(x, pltpu.MemorySpace.HBM),
        pltpu.with_memory_space_constraint(w_mat, pltpu.MemorySpace.HBM),
    )
